# baseline (speedup 1.0000x reference)
import functools

import jax
import jax.numpy as jnp
import numpy as np
from jax.experimental import pallas as pl

N = 10000
E = 160000
E_SEQ = 60000
C_S = 128
C_V = 16
C_Z = 128
H = 8
C_H = 16
P_QK = 4
P_V = 8
H_TIME = 64
SCALAR_H = 128
N_LAYERS = 2

N_PAD = 10240


def _quat_to_rot(q):
    q = q / (jnp.linalg.norm(q, axis=-1, keepdims=True) + 1e-8)
    w, x, y, z = q[..., 0], q[..., 1], q[..., 2], q[..., 3]
    R = jnp.stack([1 - 2 * (y * y + z * z), 2 * (x * y - w * z), 2 * (x * z + w * y),
                   2 * (x * y + w * z), 1 - 2 * (x * x + z * z), 2 * (y * z - w * x),
                   2 * (x * z - w * y), 2 * (y * z + w * x), 1 - 2 * (x * x + y * y)], axis=-1)
    return R.reshape(q.shape[:-1] + (3, 3))


def _ln(x):
    m = x.mean(-1, keepdims=True)
    v = ((x - m) ** 2).mean(-1, keepdims=True)
    return (x - m) / jnp.sqrt(v + 1e-5)


def _vln(v):
    n2 = (v ** 2).sum(-1).mean(-1, keepdims=True)
    return v / jnp.sqrt(n2 + 1e-5)[..., None]


def _rbf(d, n, d_max=20.0):
    mu = jnp.linspace(0.0, d_max, n)
    sigma = d_max / n
    return jnp.exp(-(((d[..., None] - mu) / sigma) ** 2))


def _posemb(diff, n):
    freq = jnp.exp(jnp.arange(0, n, 2, dtype=jnp.float32) * (-np.log(10000.0) / n))
    ang = diff[..., None].astype(jnp.float32) * freq
    return jnp.concatenate([jnp.cos(ang), jnp.sin(ang)], axis=-1)


def _edge_feats(trans, ei):
    dvec = trans[ei[0]] - trans[ei[1]]
    d = jnp.sqrt((dvec ** 2).sum(-1) + 1e-12)
    return jnp.concatenate([_rbf(d, C_Z // 2), _posemb(ei[0] - ei[1], C_Z // 2)], axis=-1)


def _edge_transition(p, s, z, ei):
    h = jnp.concatenate([s[ei[0]], s[ei[1]], z], axis=-1)
    return _ln(jax.nn.relu(h @ p['W1']) @ p['W2'])


def _point_attn(p, s, rots, trans, z, ei, v_in):
    src, dst = ei[0], ei[1]
    vn = jnp.sqrt((v_in ** 2).sum(-1) + 1e-8)
    s_eff = s + vn @ p['Wvn']
    q = (s_eff @ p['Wq']).reshape(N, H, C_H)
    k = (s_eff @ p['Wk']).reshape(N, H, C_H)
    v = (s_eff @ p['Wv']).reshape(N, H, C_H)

    def to_global(pts):
        return jnp.einsum('nij,nhpj->nhpi', rots, pts) + trans[:, None, None, :]

    qp = to_global((s_eff @ p['Wqp']).reshape(N, H, P_QK, 3))
    kp = to_global((s_eff @ p['Wkp']).reshape(N, H, P_QK, 3))
    vp = to_global((s_eff @ p['Wvp']).reshape(N, H, P_V, 3))
    logits = (q[dst] * k[src]).sum(-1) / np.sqrt(C_H) + z @ p['Wb']
    d2 = ((qp[dst] - kp[src]) ** 2).sum(axis=(-1, -2))
    gamma = jax.nn.softplus(p['head_w'])
    logits = logits / np.sqrt(3.0) - 0.5 * gamma * d2
    m = jax.lax.stop_gradient(jax.ops.segment_max(logits, dst, num_segments=N))
    a = jnp.exp(logits - m[dst])
    denom = jax.ops.segment_sum(a, dst, num_segments=N)
    a = a / (denom[dst] + 1e-9)
    o = jax.ops.segment_sum(a[..., None] * v[src], dst, num_segments=N)
    op = jax.ops.segment_sum(a[..., None, None] * vp[src], dst, num_segments=N)
    op_local = jnp.einsum('nji,nhpj->nhpi', rots, op - trans[:, None, None, :])
    opn = jnp.sqrt((op_local ** 2).sum(-1) + 1e-8)
    feat = jnp.concatenate([o.reshape(N, -1), op_local.reshape(N, -1), opn.reshape(N, -1)], axis=-1)
    ds = feat @ p['Wo_s']
    dv = jnp.einsum('nmi,mk->nki', op_local.reshape(N, H * P_V, 3), p['Wo_v'])
    return ds, dv


def _nt_body(x_ref, w1_ref, w2_ref, o_ref):
    h = jnp.maximum(jnp.dot(x_ref[...], w1_ref[...],
                            preferred_element_type=jnp.float32), 0.0)
    o_ref[...] = jnp.dot(h, w2_ref[...], preferred_element_type=jnp.float32)


@functools.partial(jax.jit, static_argnames=())
def _nt_pallas(x, w1, w2):
    xp = jnp.zeros((N_PAD, C_S), jnp.float32).at[:N].set(x)
    out = pl.pallas_call(
        _nt_body,
        grid=(N_PAD // 256,),
        in_specs=[
            pl.BlockSpec((256, C_S), lambda i: (i, 0)),
            pl.BlockSpec((C_S, 2 * C_S), lambda i: (0, 0)),
            pl.BlockSpec((2 * C_S, C_S), lambda i: (0, 0)),
        ],
        out_specs=pl.BlockSpec((256, C_S), lambda i: (i, 0)),
        out_shape=jax.ShapeDtypeStruct((N_PAD, C_S), jnp.float32),
    )(xp, w1, w2)
    return out[:N]


def kernel(quats, trans, t, noising_mask, x_mask, edge_index, seq_edge_index, params):
    rots = _quat_to_rot(quats)
    keep = (~x_mask).astype(jnp.float32)
    center = trans.mean(axis=0, keepdims=True)
    trans = (trans - center) * 0.1
    z = _edge_feats(trans, edge_index)
    zs = _edge_feats(trans, seq_edge_index)
    ft = _rbf(t, H_TIME, 1.0)
    et = jax.nn.relu(jax.nn.relu(ft @ params['Wt1'] + params['bt1']) @ params['Wt2'] + params['bt2'])
    res_pos = _posemb(jnp.arange(N), C_S)
    node = jnp.concatenate([res_pos, et, noising_mask[:, None]], axis=-1) @ params['Wemb'] + params['bemb']
    vecs = jnp.zeros((N, C_V, 3), jnp.float32)
    for lp in params['layers']:
        z = _edge_transition(lp['edge_tr'], node, z, edge_index)
        zs = _edge_transition(lp['seq_edge_tr'], node, zs, seq_edge_index)
        ds, dv = _point_attn(lp['attn_seq'], node, rots, trans, zs, seq_edge_index, vecs)
        node = _ln(node + ds * keep[:, None])
        vecs = _vln(vecs + dv * keep[:, None, None])
        ds, dv = _point_attn(lp['attn_spatial'], node, rots, trans, z, edge_index, vecs)
        node = _ln(node + ds * keep[:, None])
        vecs = _vln(vecs + dv * keep[:, None, None])
        vn = jnp.sqrt((vecs ** 2).sum(-1) + 1e-8)
        h = jax.nn.relu(jnp.concatenate([node, vn], axis=-1) @ lp['lfu_W1'])
        ds = h @ lp['lfu_W2']
        dv = jax.nn.sigmoid(node @ lp['lfu_Wg'])[..., None] * jnp.einsum('nki,kj->nji', vecs, lp['lfu_Wvm'])
        node = _ln(node + ds * keep[:, None])
        vecs = _vln(vecs + dv * keep[:, None, None])
        node = _ln(node + _nt_pallas(node, lp['nt_W1'], lp['nt_W2']))
        vecs = vecs * jax.nn.sigmoid(node @ lp['nt_Wg'])[..., None]
        node = node * keep[:, None]
        vecs = vecs * keep[:, None, None]
        u = (node * noising_mask[:, None]) @ lp['bb_Ws'] + (vecs * noising_mask[:, None, None]).reshape(N, -1) @ lp['bb_Wv']
        u = u * noising_mask[:, None]
        new_trans = trans + jnp.einsum('nij,nj->ni', rots, u[:, 3:])
        Ru = _quat_to_rot(jnp.concatenate([jnp.ones((N, 1), jnp.float32), u[:, :3]], axis=-1))
        rots = jnp.einsum('nij,njk->nik', rots, Ru)
        trans = new_trans
    trans = trans * 10.0 + center
    return node, trans, rots, vecs


# speedup vs baseline: 9.7951x; 9.7951x over previous
import functools

import jax
import jax.numpy as jnp
import numpy as np
from jax import lax
from jax.experimental import pallas as pl
from jax.experimental.pallas import tpu as pltpu
from jax.experimental.pallas import tpu_sc as plsc

N = 10000
E = 160000
E_SEQ = 60000
C_S = 128
C_V = 16
C_Z = 128
H = 8
C_H = 16
P_QK = 4
P_V = 8
H_TIME = 64
SCALAR_H = 128
N_LAYERS = 2

N_PAD = 10240

_NC, _NS = 2, 16
_NW = _NC * _NS
_CHUNK = 128
_NROW = 10240
_FH = 128


@functools.lru_cache(maxsize=None)
def _make_gather(V, D, B):
    per_w = B // _NW
    n_chunks = per_w // _CHUNK
    mesh = plsc.VectorSubcoreMesh(core_axis_name="c", subcore_axis_name="s")

    @functools.partial(
        pl.kernel, mesh=mesh,
        out_type=jax.ShapeDtypeStruct((B, D), jnp.float32),
        scratch_types=[
            pltpu.VMEM((n_chunks, _CHUNK), jnp.int32),
            pltpu.VMEM((_CHUNK, D), jnp.float32),
            pltpu.VMEM((_CHUNK, D), jnp.float32),
            pltpu.SemaphoreType.DMA,
            pltpu.SemaphoreType.DMA,
        ],
    )
    def gath(table_hbm, idx_hbm, out_hbm, idx_v, buf0, buf1, sem0, sem1):
        wid = lax.axis_index("s") * _NC + lax.axis_index("c")
        row0 = wid * n_chunks
        pltpu.sync_copy(idx_hbm.at[wid], idx_v)

        def body(j, carry):
            pltpu.async_copy(table_hbm.at[idx_v.at[j]], buf0, sem0).wait()
            pltpu.sync_copy(buf0, out_hbm.at[pl.ds((row0 + j) * _CHUNK, _CHUNK)])
            return carry

        lax.fori_loop(0, n_chunks, body, 0)

    return gath


def _sc_gather(table, idx_pad):
    B = idx_pad.shape[0]
    fn = _make_gather(table.shape[0], table.shape[1], B)
    return fn(table, idx_pad.reshape(_NW, B // (_NW * _CHUNK), _CHUNK))


@functools.lru_cache(maxsize=None)
def _make_scatter(B):
    n_chunks = B // (_NW * _CHUNK)
    rows_t = _NROW // _NS
    mesh = plsc.VectorSubcoreMesh(core_axis_name="c", subcore_axis_name="s")

    @functools.partial(
        pl.kernel, mesh=mesh,
        out_type=jax.ShapeDtypeStruct((2 * _NROW, _FH), jnp.float32),
        scratch_types=[
            pltpu.VMEM((n_chunks, _CHUNK), jnp.int32),
            pltpu.VMEM((_CHUNK, _FH), jnp.float32),
            pltpu.VMEM_SHARED((_NROW, _FH), jnp.float32),
        ],
    )
    def scat(vals_hbm, idx_hbm, zeros_hbm, out_hbm, idx_v, vals_v, acc):
        c = lax.axis_index("c")
        s = lax.axis_index("s")
        wid = s * _NC + c
        r0 = s * rows_t
        pltpu.sync_copy(zeros_hbm, acc.at[pl.ds(r0, rows_t)])
        plsc.subcore_barrier()
        row0 = wid * n_chunks
        pltpu.sync_copy(idx_hbm.at[wid], idx_v)

        def body(j, carry):
            pltpu.sync_copy(vals_hbm.at[pl.ds((row0 + j) * _CHUNK, _CHUNK)], vals_v)
            pltpu.sync_copy(vals_v, acc.at[idx_v.at[j]], add=True)
            return carry

        lax.fori_loop(0, n_chunks, body, 0)
        plsc.subcore_barrier()
        pltpu.sync_copy(acc.at[pl.ds(r0, rows_t)],
                        out_hbm.at[pl.ds(c * _NROW + r0, rows_t)])

    return scat


def _sc_scatter(vals, idx_pad):
    B = idx_pad.shape[0]
    fn = _make_scatter(B)
    zeros = jnp.zeros((_NROW // _NS, _FH), jnp.float32)
    out = fn(vals, idx_pad.reshape(_NW, B // (_NW * _CHUNK), _CHUNK), zeros)
    return out[:N] + out[_NROW:_NROW + N]


def _nt_body(x_ref, w1_ref, w2_ref, o_ref):
    h = jnp.maximum(jnp.dot(x_ref[...], w1_ref[...],
                            preferred_element_type=jnp.float32), 0.0)
    o_ref[...] = jnp.dot(h, w2_ref[...], preferred_element_type=jnp.float32)


def _nt_pallas(x, w1, w2):
    xp = jnp.zeros((N_PAD, C_S), jnp.float32).at[:N].set(x)
    out = pl.pallas_call(
        _nt_body,
        grid=(N_PAD // 256,),
        in_specs=[
            pl.BlockSpec((256, C_S), lambda i: (i, 0)),
            pl.BlockSpec((C_S, 2 * C_S), lambda i: (0, 0)),
            pl.BlockSpec((2 * C_S, C_S), lambda i: (0, 0)),
        ],
        out_specs=pl.BlockSpec((256, C_S), lambda i: (i, 0)),
        out_shape=jax.ShapeDtypeStruct((N_PAD, C_S), jnp.float32),
    )(xp, w1, w2)
    return out[:N]


def _quat_to_rot(q):
    q = q / (jnp.linalg.norm(q, axis=-1, keepdims=True) + 1e-8)
    w, x, y, z = q[..., 0], q[..., 1], q[..., 2], q[..., 3]
    R = jnp.stack([1 - 2 * (y * y + z * z), 2 * (x * y - w * z), 2 * (x * z + w * y),
                   2 * (x * y + w * z), 1 - 2 * (x * x + z * z), 2 * (y * z - w * x),
                   2 * (x * z - w * y), 2 * (y * z + w * x), 1 - 2 * (x * x + y * y)], axis=-1)
    return R.reshape(q.shape[:-1] + (3, 3))


def _ln(x):
    m = x.mean(-1, keepdims=True)
    v = ((x - m) ** 2).mean(-1, keepdims=True)
    return (x - m) / jnp.sqrt(v + 1e-5)


def _vln(v):
    n2 = (v ** 2).sum(-1).mean(-1, keepdims=True)
    return v / jnp.sqrt(n2 + 1e-5)[..., None]


def _rbf(d, n, d_max=20.0):
    mu = jnp.linspace(0.0, d_max, n)
    sigma = d_max / n
    return jnp.exp(-(((d[..., None] - mu) / sigma) ** 2))


def _posemb(diff, n):
    freq = jnp.exp(jnp.arange(0, n, 2, dtype=jnp.float32) * (-np.log(10000.0) / n))
    ang = diff[..., None].astype(jnp.float32) * freq
    return jnp.concatenate([jnp.cos(ang), jnp.sin(ang)], axis=-1)


def _pad_idx(idx, B):
    return jnp.concatenate([idx, jnp.zeros((B - idx.shape[0],), jnp.int32)])


def _edge_tr_sc(p, s, z, gsrc, gdst):
    h = jax.nn.relu(gsrc @ p['W1'][:C_S] + gdst @ p['W1'][C_S:2 * C_S]
                    + z @ p['W1'][2 * C_S:])
    return _ln(h @ p['W2'])


def _attn_sc(p, s, rots, trans, z, src_pad, dst_pad, mask_e, v_in):
    vn = jnp.sqrt((v_in ** 2).sum(-1) + 1e-8)
    s_eff = s + vn @ p['Wvn']
    q = s_eff @ p['Wq']
    k = s_eff @ p['Wk']
    v = s_eff @ p['Wv']

    def to_global(pts):
        return (jnp.einsum('nij,nhpj->nhpi', rots, pts)
                + trans[:, None, None, :])

    qp = to_global((s_eff @ p['Wqp']).reshape(N, H, P_QK, 3)).reshape(N, H * P_QK * 3)
    kp = to_global((s_eff @ p['Wkp']).reshape(N, H, P_QK, 3)).reshape(N, H * P_QK * 3)
    vp = to_global((s_eff @ p['Wvp']).reshape(N, H, P_V, 3)).reshape(N, H * P_V * 3)

    pad32 = jnp.zeros((N, 32), jnp.float32)
    dst_tab = jnp.concatenate([q, qp, pad32], axis=1)
    srcA = jnp.concatenate([k, kp, pad32], axis=1)
    srcB = jnp.concatenate([v, vp, pad32, pad32], axis=1)
    gd = _sc_gather(dst_tab, dst_pad)
    ga = _sc_gather(srcA, src_pad)
    gb = _sc_gather(srcB, src_pad)

    Bp = src_pad.shape[0]
    qk = (gd[:, :C_S] * ga[:, :C_S]).reshape(Bp, H, C_H).sum(-1) / np.sqrt(C_H)
    d2 = (((gd[:, C_S:224] - ga[:, C_S:224]) ** 2).reshape(Bp, H, P_QK * 3)).sum(-1)
    gamma = jax.nn.softplus(p['head_w'])
    logits = (qk + z @ p['Wb']) / np.sqrt(3.0) - 0.5 * gamma * d2
    m = jnp.max(logits)
    a = jnp.exp(logits - m) * mask_e[:, None]
    av = (a[:, :, None] * gb[:, :C_S].reshape(Bp, H, C_H)).reshape(Bp, C_S)
    avp = (a[:, :, None] * gb[:, C_S:320].reshape(Bp, H, P_V * 3)).reshape(Bp, H * P_V * 3)
    s0 = _sc_scatter(av, dst_pad)
    s1 = _sc_scatter(avp[:, :128], dst_pad)
    s2 = _sc_scatter(jnp.concatenate([avp[:, 128:], a,
                                      jnp.zeros((Bp, 56), jnp.float32)], axis=1),
                     dst_pad)

    den = s2[:, 64:72]
    deninv = 1.0 / jnp.maximum(den, 1e-30)
    o = s0.reshape(N, H, C_H) * deninv[:, :, None]
    op = (jnp.concatenate([s1, s2[:, :64]], axis=1).reshape(N, H, P_V * 3)
          * deninv[:, :, None]).reshape(N, H, P_V, 3)
    op_local = jnp.einsum('nji,nhpj->nhpi', rots, op - trans[:, None, None, :])
    opn = jnp.sqrt((op_local ** 2).sum(-1) + 1e-8)
    feat = jnp.concatenate([o.reshape(N, -1), op_local.reshape(N, -1),
                            opn.reshape(N, -1)], axis=-1)
    ds = feat @ p['Wo_s']
    dv = jnp.einsum('nmi,mk->nki', op_local.reshape(N, H * P_V, 3), p['Wo_v'])
    return ds, dv


def kernel(quats, trans, t, noising_mask, x_mask, edge_index, seq_edge_index, params):
    E_P = 163840
    ES_P = 61440
    src = _pad_idx(edge_index[0], E_P)
    dst = _pad_idx(edge_index[1], E_P)
    ssrc = _pad_idx(seq_edge_index[0], ES_P)
    sdst = _pad_idx(seq_edge_index[1], ES_P)
    mask_e = (jnp.arange(E_P) < E).astype(jnp.float32)
    mask_es = (jnp.arange(ES_P) < E_SEQ).astype(jnp.float32)

    rots = _quat_to_rot(quats)
    keep = (~x_mask).astype(jnp.float32)
    center = trans.mean(axis=0, keepdims=True)
    trans = (trans - center) * 0.1

    trans16 = jnp.zeros((N, 128), jnp.float32).at[:, :3].set(trans)

    def efeats(spad, dpad):
        gs = _sc_gather(trans16, spad)[:, :3]
        gdd = _sc_gather(trans16, dpad)[:, :3]
        d = jnp.sqrt(((gs - gdd) ** 2).sum(-1) + 1e-12)
        return jnp.concatenate([_rbf(d, C_Z // 2), _posemb(spad - dpad, C_Z // 2)],
                               axis=-1)

    z = efeats(src, dst)
    zs = efeats(ssrc, sdst)

    ft = _rbf(t, H_TIME, 1.0)
    et = jax.nn.relu(jax.nn.relu(ft @ params['Wt1'] + params['bt1']) @ params['Wt2'] + params['bt2'])
    res_pos = _posemb(jnp.arange(N), C_S)
    node = jnp.concatenate([res_pos, et, noising_mask[:, None]], axis=-1) @ params['Wemb'] + params['bemb']
    vecs = jnp.zeros((N, C_V, 3), jnp.float32)

    for lp in params['layers']:
        g_ns = _sc_gather(node, src)
        g_nd = _sc_gather(node, dst)
        g_nss = _sc_gather(node, ssrc)
        g_nsd = _sc_gather(node, sdst)
        z = _edge_tr_sc(lp['edge_tr'], node, z, g_ns, g_nd)
        zs = _edge_tr_sc(lp['seq_edge_tr'], node, zs, g_nss, g_nsd)
        ds, dv = _attn_sc(lp['attn_seq'], node, rots, trans, zs, ssrc, sdst, mask_es, vecs)
        node = _ln(node + ds * keep[:, None])
        vecs = _vln(vecs + dv * keep[:, None, None])
        ds, dv = _attn_sc(lp['attn_spatial'], node, rots, trans, z, src, dst, mask_e, vecs)
        node = _ln(node + ds * keep[:, None])
        vecs = _vln(vecs + dv * keep[:, None, None])
        vn = jnp.sqrt((vecs ** 2).sum(-1) + 1e-8)
        h = jax.nn.relu(jnp.concatenate([node, vn], axis=-1) @ lp['lfu_W1'])
        ds = h @ lp['lfu_W2']
        dv = jax.nn.sigmoid(node @ lp['lfu_Wg'])[..., None] * jnp.einsum('nki,kj->nji', vecs, lp['lfu_Wvm'])
        node = _ln(node + ds * keep[:, None])
        vecs = _vln(vecs + dv * keep[:, None, None])
        node = _ln(node + _nt_pallas(node, lp['nt_W1'], lp['nt_W2']))
        vecs = vecs * jax.nn.sigmoid(node @ lp['nt_Wg'])[..., None]
        node = node * keep[:, None]
        vecs = vecs * keep[:, None, None]
        u = (node * noising_mask[:, None]) @ lp['bb_Ws'] + (vecs * noising_mask[:, None, None]).reshape(N, -1) @ lp['bb_Wv']
        u = u * noising_mask[:, None]
        new_trans = trans + jnp.einsum('nij,nj->ni', rots, u[:, 3:])
        Ru = _quat_to_rot(jnp.concatenate([jnp.ones((N, 1), jnp.float32), u[:, :3]], axis=-1))
        rots = jnp.einsum('nij,njk->nik', rots, Ru)
        trans = new_trans
    trans = trans * 10.0 + center
    return node, trans, rots, vecs
